# Initial kernel scaffold; baseline (speedup 1.0000x reference)
import jax
import jax.numpy as jnp
from jax.experimental import pallas as pl


def kernel(inputs, W_router, b_router, W_experts, b_experts):
    raise NotImplementedError("write your pallas kernel here")


# speedup vs baseline: 2.3131x; 2.3131x over previous
import functools

import jax
import jax.numpy as jnp
from jax.experimental import pallas as pl
from jax.experimental.pallas import tpu as pltpu

_TOPK = 2


def _router_gate_body(x_ref, wr_ref, br_ref, we_ref, be_ref,
                      probs_ref, sel_ref, g_ref, const_ref):
    B, D = x_ref.shape
    E = wr_ref.shape[1]
    x = x_ref[...]
    scores = jax.lax.dot(x, wr_ref[...],
                         preferred_element_type=jnp.float32)
    scores = scores + br_ref[...]

    iota_e = jax.lax.broadcasted_iota(jnp.int32, (B, E), 1)
    rank = jnp.zeros((B, E), jnp.float32)
    for ep in range(E):
        sp = scores[:, ep:ep + 1]
        gt = (sp > scores).astype(jnp.float32)
        eq = jnp.logical_and(sp == scores, ep < iota_e).astype(jnp.float32)
        rank = rank + gt + eq
    sel = (rank < float(_TOPK)).astype(jnp.float32)
    sel_ref[...] = sel

    neg = jnp.float32(-1e30)
    v1 = jnp.max(jnp.where(rank == 0.0, scores, neg), axis=1, keepdims=True)
    v2 = jnp.max(jnp.where(rank == 1.0, scores, neg), axis=1, keepdims=True)
    p1 = 1.0 / (1.0 + jnp.exp(v2 - v1))
    probs_ref[...] = jnp.concatenate([p1, 1.0 - p1], axis=1)

    s = jax.lax.dot_general(sel, x, (((0,), (0,)), ((), ())),
                            preferred_element_type=jnp.float32)

    inv_b = jnp.float32(1.0 / B)
    const = jnp.zeros((1, we_ref.shape[2]), jnp.float32)
    for e in range(E):
        mean_e = jax.lax.dot(s[e:e + 1, :], we_ref[e],
                             preferred_element_type=jnp.float32) * inv_b
        mean_e = mean_e + be_ref[e:e + 1, :]
        m = jnp.max(mean_e, axis=1, keepdims=True)
        ex = jnp.exp(mean_e - m)
        g_e = ex / jnp.sum(ex, axis=1, keepdims=True)
        g_ref[e:e + 1, :] = g_e
        const = const + be_ref[e:e + 1, :] * g_e
    const_ref[...] = const


def _main_body(x_ref, we_ref, sel_ref, g_ref, const_ref, acc_ref):
    e = pl.program_id(1)
    TB = x_ref.shape[0]
    E = sel_ref.shape[1]
    xw = jax.lax.dot(x_ref[...], we_ref[0],
                     preferred_element_type=jnp.float32)
    iota8 = jax.lax.broadcasted_iota(jnp.int32, (TB, E), 1)
    selcol = jnp.sum(sel_ref[...] * (iota8 == e).astype(jnp.float32),
                     axis=1, keepdims=True)
    iotag = jax.lax.broadcasted_iota(jnp.int32, (E, 1), 0)
    grow = jnp.sum(g_ref[...] * (iotag == e).astype(jnp.float32),
                   axis=0, keepdims=True)
    contrib = xw * grow * selcol

    @pl.when(e == 0)
    def _():
        acc_ref[...] = const_ref[...] + contrib

    @pl.when(e != 0)
    def _():
        acc_ref[...] = acc_ref[...] + contrib


def kernel(inputs, W_router, b_router, W_experts, b_experts):
    B, D = inputs.shape
    E, _, OUT = W_experts.shape
    br2 = b_router.reshape(1, E)

    probs, sel, g, const = pl.pallas_call(
        _router_gate_body,
        out_shape=[
            jax.ShapeDtypeStruct((B, _TOPK), jnp.float32),
            jax.ShapeDtypeStruct((B, E), jnp.float32),
            jax.ShapeDtypeStruct((E, OUT), jnp.float32),
            jax.ShapeDtypeStruct((1, OUT), jnp.float32),
        ],
    )(inputs, W_router, br2, W_experts, b_experts)

    TB = 512
    grid = (B // TB, E)
    final = pl.pallas_call(
        _main_body,
        grid=grid,
        in_specs=[
            pl.BlockSpec((TB, D), lambda b, e: (b, 0)),
            pl.BlockSpec((1, D, OUT), lambda b, e: (e, 0, 0)),
            pl.BlockSpec((TB, E), lambda b, e: (b, 0)),
            pl.BlockSpec((E, OUT), lambda b, e: (0, 0)),
            pl.BlockSpec((1, OUT), lambda b, e: (0, 0)),
        ],
        out_specs=pl.BlockSpec((TB, OUT), lambda b, e: (b, 0)),
        out_shape=jax.ShapeDtypeStruct((B, OUT), jnp.float32),
        compiler_params=pltpu.CompilerParams(
            dimension_semantics=("arbitrary", "arbitrary"),
        ),
    )(inputs, W_experts, sel, g, const)

    return final, probs


# speedup vs baseline: 3.4429x; 1.4884x over previous
import jax
import jax.numpy as jnp
from jax.experimental import pallas as pl
from jax.experimental.pallas import tpu as pltpu

_TOPK = 2


def _fused_body(x_ref, wr_ref, br_ref, we_ref, be_ref,
                final_ref, probs_ref,
                sel_ref, s_ref, const_ref):
    e = pl.program_id(0)
    E = pl.num_programs(0)
    B, D = x_ref.shape
    OUT = we_ref.shape[2]
    x = x_ref[...]

    @pl.when(e == 0)
    def _():
        scores = jax.lax.dot(x, wr_ref[...],
                             preferred_element_type=jnp.float32)
        scores = scores + br_ref[...]
        iota_e = jax.lax.broadcasted_iota(jnp.int32, (B, E), 1)
        rank = jnp.zeros((B, E), jnp.float32)
        for ep in range(8):
            sp = scores[:, ep:ep + 1]
            gt = (sp > scores).astype(jnp.float32)
            eq = jnp.logical_and(sp == scores, ep < iota_e).astype(jnp.float32)
            rank = rank + gt + eq
        sel = (rank < float(_TOPK)).astype(jnp.float32)
        sel_ref[...] = sel

        neg = jnp.float32(-1e30)
        v1 = jnp.max(jnp.where(rank == 0.0, scores, neg), axis=1,
                     keepdims=True)
        v2 = jnp.max(jnp.where(rank == 1.0, scores, neg), axis=1,
                     keepdims=True)
        p1 = 1.0 / (1.0 + jnp.exp(v2 - v1))
        probs_ref[...] = jnp.concatenate([p1, 1.0 - p1], axis=1)

        s_ref[...] = jax.lax.dot_general(
            sel, x, (((0,), (0,)), ((), ())),
            preferred_element_type=jnp.float32)
        const_ref[...] = jnp.zeros((1, OUT), jnp.float32)

    s_row = s_ref[pl.ds(e, 1), :]
    iota_be = jax.lax.broadcasted_iota(jnp.int32, (E, 1), 0)
    be_row = jnp.sum(be_ref[...] * (iota_be == e).astype(jnp.float32),
                     axis=0, keepdims=True)
    mean_e = jax.lax.dot(s_row, we_ref[0],
                         preferred_element_type=jnp.float32)
    mean_e = mean_e * jnp.float32(1.0 / B) + be_row
    m = jnp.max(mean_e, axis=1, keepdims=True)
    ex = jnp.exp(mean_e - m)
    g_e = ex / jnp.sum(ex, axis=1, keepdims=True)
    const_ref[...] += be_row * g_e

    iota8 = jax.lax.broadcasted_iota(jnp.int32, (B, E), 1)
    selcol = jnp.sum(sel_ref[...] * (iota8 == e).astype(jnp.float32),
                     axis=1, keepdims=True)
    contrib = jax.lax.dot(x, we_ref[0],
                          preferred_element_type=jnp.float32) * g_e * selcol

    @pl.when(e == 0)
    def _():
        final_ref[...] = contrib

    @pl.when(e != 0)
    def _():
        final_ref[...] = final_ref[...] + contrib

    @pl.when(e == E - 1)
    def _():
        final_ref[...] = final_ref[...] + const_ref[...]


def kernel(inputs, W_router, b_router, W_experts, b_experts):
    B, D = inputs.shape
    E, _, OUT = W_experts.shape
    br2 = b_router.reshape(1, E)

    final, probs = pl.pallas_call(
        _fused_body,
        grid=(E,),
        in_specs=[
            pl.BlockSpec((B, D), lambda e: (0, 0)),
            pl.BlockSpec((D, E), lambda e: (0, 0)),
            pl.BlockSpec((1, E), lambda e: (0, 0)),
            pl.BlockSpec((1, D, OUT), lambda e: (e, 0, 0)),
            pl.BlockSpec((E, OUT), lambda e: (0, 0)),
        ],
        out_specs=[
            pl.BlockSpec((B, OUT), lambda e: (0, 0)),
            pl.BlockSpec((B, _TOPK), lambda e: (0, 0)),
        ],
        out_shape=[
            jax.ShapeDtypeStruct((B, OUT), jnp.float32),
            jax.ShapeDtypeStruct((B, _TOPK), jnp.float32),
        ],
        scratch_shapes=[
            pltpu.VMEM((B, E), jnp.float32),
            pltpu.VMEM((E, D), jnp.float32),
            pltpu.VMEM((1, OUT), jnp.float32),
        ],
        compiler_params=pltpu.CompilerParams(
            dimension_semantics=("arbitrary",),
        ),
    )(inputs, W_router, br2, W_experts, b_experts)

    return final, probs
